# baseline (speedup 1.0000x reference)
import jax
import jax.numpy as jnp
from jax.experimental import pallas as pl
from jax.experimental.pallas import tpu as pltpu

_B = 256
_K = 28224
_F = 1500

_FT = 768
_KT = 4096
_KB = 7

_IT = 2048
_IB = 14


def _enc_kernel(x_ref, w_ref, be_ref, y_ref, jac_ref, acc_ref, rn2_ref):
    f = pl.program_id(0)
    k = pl.program_id(1)

    @pl.when(k == 0)
    def _init():
        acc_ref[...] = jnp.zeros_like(acc_ref)
        rn2_ref[...] = jnp.zeros_like(rn2_ref)

    lane = jax.lax.broadcasted_iota(jnp.int32, (1, _KT), 1)
    valid = (k * _KT + lane) < _K
    xb = jnp.where(valid, x_ref[...], 0.0)
    wb = jnp.where(valid, w_ref[...], 0.0)

    acc_ref[...] += jax.lax.dot_general(
        xb, wb, (((1,), (1,)), ((), ())),
        preferred_element_type=jnp.float32)
    rn2_ref[...] += jnp.sum(wb * wb, axis=1, keepdims=True)

    @pl.when(k == _KB - 1)
    def _finish():
        y = jax.nn.sigmoid(acc_ref[...] + be_ref[...])
        y_ref[...] = y
        s = y * (1.0 - y)
        s2c = jnp.sum(s * s, axis=0, keepdims=True)
        rn2_row = rn2_ref[...].T
        col = jax.lax.broadcasted_iota(jnp.int32, (1, _FT), 1)
        fvalid = (f * _FT + col) < _F
        val = jnp.sum(jnp.where(fvalid, s2c * rn2_row, 0.0), keepdims=True)
        jac_ref[...] = val.reshape(1, 1, 1)


def _dec_kernel(y_ref, w_ref, bd_ref, o_ref):
    o_ref[...] = jax.nn.sigmoid(
        jax.lax.dot_general(
            y_ref[...], w_ref[...], (((1,), (1,)), ((), ())),
            preferred_element_type=jnp.float32)
        + bd_ref[...])


def kernel(x, W_enc, b_enc, W_dec, b_dec):
    y_enc, jac_parts = pl.pallas_call(
        _enc_kernel,
        grid=(2, _KB),
        in_specs=[
            pl.BlockSpec((_B, _KT), lambda f, k: (0, k)),
            pl.BlockSpec((_FT, _KT), lambda f, k: (f, k)),
            pl.BlockSpec((1, _FT), lambda f, k: (0, f)),
        ],
        out_specs=[
            pl.BlockSpec((_B, _FT), lambda f, k: (0, f)),
            pl.BlockSpec((1, 1, 1), lambda f, k: (f, 0, 0)),
        ],
        out_shape=[
            jax.ShapeDtypeStruct((_B, _F), jnp.float32),
            jax.ShapeDtypeStruct((2, 1, 1), jnp.float32),
        ],
        scratch_shapes=[
            pltpu.VMEM((_B, _FT), jnp.float32),
            pltpu.VMEM((_FT, 1), jnp.float32),
        ],
        compiler_params=pltpu.CompilerParams(
            dimension_semantics=("arbitrary", "arbitrary")),
    )(x, W_enc, b_enc.reshape(1, _F))

    jac_reg = jnp.sum(jac_parts)

    y_out = pl.pallas_call(
        _dec_kernel,
        grid=(_IB,),
        in_specs=[
            pl.BlockSpec((_B, _F), lambda i: (0, 0)),
            pl.BlockSpec((_IT, _F), lambda i: (i, 0)),
            pl.BlockSpec((1, _IT), lambda i: (0, i)),
        ],
        out_specs=pl.BlockSpec((_B, _IT), lambda i: (0, i)),
        out_shape=jax.ShapeDtypeStruct((_B, _K), jnp.float32),
        compiler_params=pltpu.CompilerParams(
            dimension_semantics=("arbitrary",)),
    )(y_enc, W_dec, b_dec.reshape(1, _K))

    return y_out, jac_reg


# speedup vs baseline: 1.0254x; 1.0254x over previous
import jax
import jax.numpy as jnp
from jax.experimental import pallas as pl
from jax.experimental.pallas import tpu as pltpu

_B = 256
_K = 28224
_F = 1500

_FP = 1536
_FS = 256
_NWE = 6
_KT = 2048
_KB = 14

_IT = 2048
_IS = 512
_NWD = 4
_IB = 14


def _enc_kernel(x_ref, *refs):
    w_refs = refs[:_NWE]
    be_ref = refs[_NWE]
    y_ref, jac_ref, acc_ref, rn2_ref = refs[_NWE + 1:]
    k = pl.program_id(0)

    @pl.when(k == 0)
    def _init():
        acc_ref[...] = jnp.zeros_like(acc_ref)
        rn2_ref[...] = jnp.zeros_like(rn2_ref)

    lane = jax.lax.broadcasted_iota(jnp.int32, (1, _KT), 1)
    valid = (k * _KT + lane) < _K
    xb = jnp.where(valid, x_ref[...], 0.0)
    for i in range(_NWE):
        wb = jnp.where(valid, w_refs[i][...], 0.0)
        acc_ref[:, i * _FS:(i + 1) * _FS] += jax.lax.dot_general(
            xb, wb, (((1,), (1,)), ((), ())),
            preferred_element_type=jnp.float32)
        rn2_ref[i * _FS:(i + 1) * _FS, :] += jnp.sum(
            wb * wb, axis=1, keepdims=True)

    @pl.when(k == _KB - 1)
    def _finish():
        y = jax.nn.sigmoid(acc_ref[:, :_F] + be_ref[...])
        y_ref[...] = y
        s = y * (1.0 - y)
        s2c = jnp.sum(s * s, axis=0, keepdims=True)
        rn2_row = rn2_ref[...].T[:, :_F]
        val = jnp.sum(s2c * rn2_row, keepdims=True)
        jac_ref[...] = val.reshape(1, 1, 1)


def _dec_kernel(y_ref, *refs):
    w_refs = refs[:_NWD]
    bd_ref = refs[_NWD]
    o_ref = refs[_NWD + 1]
    for j in range(_NWD):
        o_ref[:, j * _IS:(j + 1) * _IS] = jax.nn.sigmoid(
            jax.lax.dot_general(
                y_ref[...], w_refs[j][...], (((1,), (1,)), ((), ())),
                preferred_element_type=jnp.float32)
            + bd_ref[:, j * _IS:(j + 1) * _IS])


def kernel(x, W_enc, b_enc, W_dec, b_dec):
    w_enc_specs = [
        pl.BlockSpec((_FS, _KT), lambda k, i=i: (i, k)) for i in range(_NWE)
    ]
    y_enc, jac_parts = pl.pallas_call(
        _enc_kernel,
        grid=(_KB,),
        in_specs=[pl.BlockSpec((_B, _KT), lambda k: (0, k))]
        + w_enc_specs
        + [pl.BlockSpec((1, _F), lambda k: (0, 0))],
        out_specs=[
            pl.BlockSpec((_B, _F), lambda k: (0, 0)),
            pl.BlockSpec((1, 1, 1), lambda k: (0, 0, 0)),
        ],
        out_shape=[
            jax.ShapeDtypeStruct((_B, _F), jnp.float32),
            jax.ShapeDtypeStruct((1, 1, 1), jnp.float32),
        ],
        scratch_shapes=[
            pltpu.VMEM((_B, _FP), jnp.float32),
            pltpu.VMEM((_FP, 1), jnp.float32),
        ],
        compiler_params=pltpu.CompilerParams(
            dimension_semantics=("arbitrary",)),
    )(x, *([W_enc] * _NWE), b_enc.reshape(1, _F))

    jac_reg = jnp.sum(jac_parts)

    w_dec_specs = [
        pl.BlockSpec((_IS, _F), lambda i, j=j: (_NWD * i + j, 0))
        for j in range(_NWD)
    ]
    y_out = pl.pallas_call(
        _dec_kernel,
        grid=(_IB,),
        in_specs=[pl.BlockSpec((_B, _F), lambda i: (0, 0))]
        + w_dec_specs
        + [pl.BlockSpec((1, _IT), lambda i: (0, i))],
        out_specs=pl.BlockSpec((_B, _IT), lambda i: (0, i)),
        out_shape=jax.ShapeDtypeStruct((_B, _K), jnp.float32),
        compiler_params=pltpu.CompilerParams(
            dimension_semantics=("arbitrary",)),
    )(y_enc, *([W_dec] * _NWD), b_dec.reshape(1, _K))

    return y_out, jac_reg


# speedup vs baseline: 2.2632x; 2.2072x over previous
import jax
import jax.numpy as jnp
from jax.experimental import pallas as pl
from jax.experimental.pallas import tpu as pltpu

_B = 256
_K = 28224
_F = 1500

_FP = 1536
_FS = 256
_NWE = 6
_KT = 2048
_KB = 14

_IT = 2048
_IB = 14


def _enc_kernel(xt_ref, *refs):
    w_refs = refs[:_NWE]
    be_ref = refs[_NWE]
    y_ref, jac_ref, acc_ref, rn2_ref = refs[_NWE + 1:]
    k = pl.program_id(0)

    @pl.when(k == 0)
    def _init():
        acc_ref[...] = jnp.zeros_like(acc_ref)
        rn2_ref[...] = jnp.zeros_like(rn2_ref)

    row = jax.lax.broadcasted_iota(jnp.int32, (_KT, 1), 0)
    lane = jax.lax.broadcasted_iota(jnp.int32, (1, _KT), 1)
    xb = jnp.where(k * _KT + row < _K, xt_ref[...], 0.0)
    for i in range(_NWE):
        wb = jnp.where(k * _KT + lane < _K, w_refs[i][...], 0.0)
        acc_ref[i * _FS:(i + 1) * _FS, :] += jax.lax.dot_general(
            wb, xb, (((1,), (0,)), ((), ())),
            preferred_element_type=jnp.float32)
        rn2_ref[i * _FS:(i + 1) * _FS, :] += jnp.sum(
            wb * wb, axis=1, keepdims=True)

    @pl.when(k == _KB - 1)
    def _finish():
        y = jax.nn.sigmoid(acc_ref[:_F, :] + be_ref[...])
        y_ref[...] = y
        s = y * (1.0 - y)
        s2r = jnp.sum(s * s, axis=1, keepdims=True)
        val = jnp.sum(s2r * rn2_ref[:_F, :], keepdims=True)
        jac_ref[...] = val.reshape(1, 1, 1)


def _dec_kernel(y_ref, wt_ref, bd_ref, o_ref):
    o_ref[...] = jax.nn.sigmoid(
        jax.lax.dot_general(
            wt_ref[...], y_ref[...], (((0,), (0,)), ((), ())),
            preferred_element_type=jnp.float32)
        + bd_ref[...])


def kernel(x, W_enc, b_enc, W_dec, b_dec):
    xt = x.T
    w_dec_t = W_dec.T

    w_enc_specs = [
        pl.BlockSpec((_FS, _KT), lambda k, i=i: (i, k)) for i in range(_NWE)
    ]
    y_enc_t, jac_parts = pl.pallas_call(
        _enc_kernel,
        grid=(_KB,),
        in_specs=[pl.BlockSpec((_KT, _B), lambda k: (k, 0))]
        + w_enc_specs
        + [pl.BlockSpec((_F, 1), lambda k: (0, 0))],
        out_specs=[
            pl.BlockSpec((_F, _B), lambda k: (0, 0)),
            pl.BlockSpec((1, 1, 1), lambda k: (0, 0, 0)),
        ],
        out_shape=[
            jax.ShapeDtypeStruct((_F, _B), jnp.float32),
            jax.ShapeDtypeStruct((1, 1, 1), jnp.float32),
        ],
        scratch_shapes=[
            pltpu.VMEM((_FP, _B), jnp.float32),
            pltpu.VMEM((_FP, 1), jnp.float32),
        ],
        compiler_params=pltpu.CompilerParams(
            dimension_semantics=("arbitrary",)),
    )(xt, *([W_enc] * _NWE), b_enc.reshape(_F, 1))

    jac_reg = jnp.sum(jac_parts)

    y_out_t = pl.pallas_call(
        _dec_kernel,
        grid=(_IB,),
        in_specs=[
            pl.BlockSpec((_F, _B), lambda i: (0, 0)),
            pl.BlockSpec((_F, _IT), lambda i: (0, i)),
            pl.BlockSpec((_IT, 1), lambda i: (i, 0)),
        ],
        out_specs=pl.BlockSpec((_IT, _B), lambda i: (i, 0)),
        out_shape=jax.ShapeDtypeStruct((_K, _B), jnp.float32),
        compiler_params=pltpu.CompilerParams(
            dimension_semantics=("arbitrary",)),
    )(y_enc_t, w_dec_t, b_dec.reshape(_K, 1))

    return y_out_t.T, jac_reg


# speedup vs baseline: 2.2666x; 1.0015x over previous
import jax
import jax.numpy as jnp
from jax.experimental import pallas as pl
from jax.experimental.pallas import tpu as pltpu

_B = 256
_K = 28224
_F = 1500

_FP = 1536
_FS = 256
_NWE = 6
_KT = 2048
_KB = 14

_IT = 2048
_IB = 14


def _enc_kernel(xt_ref, *refs):
    w_refs = refs[:_NWE]
    be_ref = refs[_NWE]
    y_ref, jac_ref, acc_ref, rn2_ref = refs[_NWE + 1:]
    k = pl.program_id(0)

    @pl.when(k == 0)
    def _init():
        acc_ref[...] = jnp.zeros_like(acc_ref)
        rn2_ref[...] = jnp.zeros_like(rn2_ref)

    row = jax.lax.broadcasted_iota(jnp.int32, (_KT, 1), 0)
    lane = jax.lax.broadcasted_iota(jnp.int32, (1, _KT), 1)
    xb = jnp.where(k * _KT + row < _K, xt_ref[...], 0.0)
    for i in range(_NWE):
        wb = jnp.where(k * _KT + lane < _K, w_refs[i][...], 0.0)
        acc_ref[i * _FS:(i + 1) * _FS, :] += jax.lax.dot_general(
            wb, xb, (((1,), (0,)), ((), ())),
            preferred_element_type=jnp.float32)
        rn2_ref[i * _FS:(i + 1) * _FS, :] += jnp.sum(
            wb * wb, axis=1, keepdims=True)

    @pl.when(k == _KB - 1)
    def _finish():
        y = jax.nn.sigmoid(acc_ref[:_F, :] + be_ref[...])
        y_ref[...] = y
        s = y * (1.0 - y)
        s2r = jnp.sum(s * s, axis=1, keepdims=True)
        val = jnp.sum(s2r * rn2_ref[:_F, :], keepdims=True)
        jac_ref[...] = val.reshape(1, 1, 1)


def _dec_kernel(y_ref, wt_ref, bd_ref, o_ref):
    o_ref[...] = jax.nn.sigmoid(
        jax.lax.dot_general(
            wt_ref[...], y_ref[...], (((0,), (0,)), ((), ())),
            preferred_element_type=jnp.float32)
        + bd_ref[...])


def kernel(x, W_enc, b_enc, W_dec, b_dec):
    xt = x.T
    w_dec_t = W_dec.T

    w_enc_specs = [
        pl.BlockSpec((_FS, _KT), lambda k, i=i: (i, k)) for i in range(_NWE)
    ]
    y_enc_t, jac_parts = pl.pallas_call(
        _enc_kernel,
        grid=(_KB,),
        in_specs=[pl.BlockSpec((_KT, _B), lambda k: (k, 0))]
        + w_enc_specs
        + [pl.BlockSpec((_F, 1), lambda k: (0, 0))],
        out_specs=[
            pl.BlockSpec((_F, _B), lambda k: (0, 0)),
            pl.BlockSpec((1, 1, 1), lambda k: (0, 0, 0)),
        ],
        out_shape=[
            jax.ShapeDtypeStruct((_F, _B), jnp.float32),
            jax.ShapeDtypeStruct((1, 1, 1), jnp.float32),
        ],
        scratch_shapes=[
            pltpu.VMEM((_FP, _B), jnp.float32),
            pltpu.VMEM((_FP, 1), jnp.float32),
        ],
        compiler_params=pltpu.CompilerParams(
            dimension_semantics=("arbitrary",)),
    )(xt, *([W_enc] * _NWE), b_enc.reshape(_F, 1))

    jac_reg = jac_parts.reshape(())

    y_out_t = pl.pallas_call(
        _dec_kernel,
        grid=(_IB,),
        in_specs=[
            pl.BlockSpec((_F, _B), lambda i: (0, 0)),
            pl.BlockSpec((_F, _IT), lambda i: (0, i)),
            pl.BlockSpec((_IT, 1), lambda i: (i, 0)),
        ],
        out_specs=pl.BlockSpec((_IT, _B), lambda i: (i, 0)),
        out_shape=jax.ShapeDtypeStruct((_K, _B), jnp.float32),
        compiler_params=pltpu.CompilerParams(
            dimension_semantics=("arbitrary",)),
    )(y_enc_t, w_dec_t, b_dec.reshape(_K, 1))

    return y_out_t.T, jac_reg


# speedup vs baseline: 2.2923x; 1.0113x over previous
import jax
import jax.numpy as jnp
from jax.experimental import pallas as pl
from jax.experimental.pallas import tpu as pltpu

_B = 256
_K = 28224
_F = 1500

_FP = 1536
_FS = 256
_NWE = 6
_KT = 2048
_KB = 14

_IT = 2048
_IB = 14


def _enc_kernel(xt_ref, *refs):
    w_refs = refs[:_NWE]
    be_ref = refs[_NWE]
    y_ref, jac_ref, acc_ref, rn2_ref = refs[_NWE + 1:]
    k = pl.program_id(0)

    @pl.when(k == 0)
    def _init():
        acc_ref[...] = jnp.zeros_like(acc_ref)
        rn2_ref[...] = jnp.zeros_like(rn2_ref)

    row = jax.lax.broadcasted_iota(jnp.int32, (_KT, 1), 0)
    lane = jax.lax.broadcasted_iota(jnp.int32, (1, _KT), 1)
    xb = jnp.where(k * _KT + row < _K, xt_ref[...], 0.0)
    for i in range(_NWE):
        wb = jnp.where(k * _KT + lane < _K, w_refs[i][...], 0.0)
        acc_ref[i * _FS:(i + 1) * _FS, :] += jax.lax.dot_general(
            wb, xb, (((1,), (0,)), ((), ())),
            preferred_element_type=jnp.float32)
        rn2_ref[i * _FS:(i + 1) * _FS, :] += jnp.sum(
            wb * wb, axis=1, keepdims=True)

    @pl.when(k == _KB - 1)
    def _finish():
        y = jax.nn.sigmoid(acc_ref[:_F, :] + be_ref[...])
        y_ref[...] = y
        s = y * (1.0 - y)
        s2r = jnp.sum(s * s, axis=1, keepdims=True)
        val = jnp.sum(s2r * rn2_ref[:_F, :], keepdims=True)
        jac_ref[...] = val.reshape(1, 1, 1)


_NWD = 4
_IS = _IT // _NWD


def _dec_kernel(y_ref, *refs):
    w_refs = refs[:_NWD]
    bd_ref = refs[_NWD]
    o_ref = refs[_NWD + 1]
    for j in range(_NWD):
        o_ref[j * _IS:(j + 1) * _IS, :] = jax.nn.sigmoid(
            jax.lax.dot_general(
                w_refs[j][...], y_ref[...], (((0,), (0,)), ((), ())),
                preferred_element_type=jnp.float32)
            + bd_ref[j * _IS:(j + 1) * _IS, :])


def kernel(x, W_enc, b_enc, W_dec, b_dec):
    xt = x.T
    w_dec_t = W_dec.T

    w_enc_specs = [
        pl.BlockSpec((_FS, _KT), lambda k, i=i: (i, k)) for i in range(_NWE)
    ]
    y_enc_t, jac_parts = pl.pallas_call(
        _enc_kernel,
        grid=(_KB,),
        in_specs=[pl.BlockSpec((_KT, _B), lambda k: (k, 0))]
        + w_enc_specs
        + [pl.BlockSpec((_F, 1), lambda k: (0, 0))],
        out_specs=[
            pl.BlockSpec((_F, _B), lambda k: (0, 0)),
            pl.BlockSpec((1, 1, 1), lambda k: (0, 0, 0)),
        ],
        out_shape=[
            jax.ShapeDtypeStruct((_F, _B), jnp.float32),
            jax.ShapeDtypeStruct((1, 1, 1), jnp.float32),
        ],
        scratch_shapes=[
            pltpu.VMEM((_FP, _B), jnp.float32),
            pltpu.VMEM((_FP, 1), jnp.float32),
        ],
        compiler_params=pltpu.CompilerParams(
            dimension_semantics=("arbitrary",)),
    )(xt, *([W_enc] * _NWE), b_enc.reshape(_F, 1))

    jac_reg = jac_parts.reshape(())

    y_out_t = pl.pallas_call(
        _dec_kernel,
        grid=(_IB,),
        in_specs=[pl.BlockSpec((_F, _B), lambda i: (0, 0))]
        + [pl.BlockSpec((_F, _IS), lambda i, j=j: (0, _NWD * i + j))
           for j in range(_NWD)]
        + [pl.BlockSpec((_IT, 1), lambda i: (i, 0))],
        out_specs=pl.BlockSpec((_IT, _B), lambda i: (i, 0)),
        out_shape=jax.ShapeDtypeStruct((_K, _B), jnp.float32),
        compiler_params=pltpu.CompilerParams(
            dimension_semantics=("arbitrary",)),
    )(y_enc_t, *([w_dec_t] * _NWD), b_dec.reshape(_K, 1))

    return y_out_t.T, jac_reg
